# baseline (speedup 1.0000x reference)
import jax
import jax.numpy as jnp
from jax.experimental import pallas as pl
from jax.experimental.pallas import tpu as pltpu

_BM = 128


def _gcn_body(x_ref, adj_ref, w1_ref, w2_ref, w3_ref, b1_ref, b2_ref, b3_ref,
              o_ref, adjbf_ref, s1_ref, s2_ref, s3_ref):
    l = pl.program_id(0)
    i = pl.program_id(1)
    bf = jnp.bfloat16
    rows = pl.ds(i * _BM, _BM)

    @pl.when(l == 0)
    def _phase0():
        s1_ref[rows, :] = jnp.dot(
            x_ref[...].astype(bf), w1_ref[...].astype(bf),
            preferred_element_type=jnp.float32).astype(bf)

    @pl.when(l == 1)
    def _phase1():
        ab = adj_ref[...].astype(bf)
        adjbf_ref[rows, :] = ab
        agg = jnp.dot(ab, s1_ref[...], preferred_element_type=jnp.float32)
        h = jnp.maximum(agg + b1_ref[...], 0.0)
        s2_ref[rows, :] = jnp.dot(
            h.astype(bf), w2_ref[...].astype(bf),
            preferred_element_type=jnp.float32).astype(bf)

    @pl.when(l == 2)
    def _phase2():
        agg = jnp.dot(adjbf_ref[rows, :], s2_ref[...],
                      preferred_element_type=jnp.float32)
        h = jnp.maximum(agg + b2_ref[...], 0.0)
        s3_ref[rows, :] = jnp.dot(
            h.astype(bf), w3_ref[...].astype(bf),
            preferred_element_type=jnp.float32).astype(bf)

    @pl.when(l == 3)
    def _phase3():
        agg = jnp.dot(adjbf_ref[rows, :], s3_ref[...],
                      preferred_element_type=jnp.float32)
        o_ref[...] = jnp.maximum(agg + b3_ref[...], 0.0)


def kernel(x, adj, W1, b1, W2, b2, W3, b3):
    n, d_in = x.shape
    c1 = W1.shape[1]
    c2 = W2.shape[1]
    c3 = W3.shape[1]
    nb = n // _BM

    return pl.pallas_call(
        _gcn_body,
        grid=(4, nb),
        in_specs=[
            pl.BlockSpec((_BM, d_in), lambda l, i: (jnp.where(l == 0, i, 0), 0)),
            pl.BlockSpec((_BM, n), lambda l, i: (jnp.where(l == 1, i, 0), 0)),
            pl.BlockSpec((d_in, c1), lambda l, i: (0, 0)),
            pl.BlockSpec((c1, c2), lambda l, i: (0, 0)),
            pl.BlockSpec((c2, c3), lambda l, i: (0, 0)),
            pl.BlockSpec((1, c1), lambda l, i: (0, 0)),
            pl.BlockSpec((1, c2), lambda l, i: (0, 0)),
            pl.BlockSpec((1, c3), lambda l, i: (0, 0)),
        ],
        out_specs=pl.BlockSpec((_BM, c3), lambda l, i: (jnp.where(l == 3, i, 0), 0)),
        out_shape=jax.ShapeDtypeStruct((n, c3), jnp.float32),
        scratch_shapes=[
            pltpu.VMEM((n, n), jnp.bfloat16),
            pltpu.VMEM((n, c1), jnp.bfloat16),
            pltpu.VMEM((n, c2), jnp.bfloat16),
            pltpu.VMEM((n, c3), jnp.bfloat16),
        ],
    )(x, adj, W1, W2, W3,
      b1.reshape(1, -1), b2.reshape(1, -1), b3.reshape(1, -1))


# speedup vs baseline: 1.1009x; 1.1009x over previous
import jax
import jax.numpy as jnp
from jax.experimental import pallas as pl
from jax.experimental.pallas import tpu as pltpu

_BM = 128


def _gcn_body(x_ref, adj_ref, w1_ref, w2_ref, w3_ref, b1_ref, b2_ref, b3_ref,
              o_ref, adjbf_ref, s2_ref, s3_ref):
    l = pl.program_id(0)
    i = pl.program_id(1)
    bf = jnp.bfloat16
    rows = pl.ds(i * _BM, _BM)

    @pl.when(l == 0)
    def _phase0():
        ab = adj_ref[...].astype(bf)
        adjbf_ref[rows, :] = ab
        aggx = jnp.dot(ab, x_ref[...], preferred_element_type=jnp.float32)
        t = jnp.dot(aggx.astype(bf), w1_ref[...].astype(bf),
                    preferred_element_type=jnp.float32)
        h = jnp.maximum(t + b1_ref[...], 0.0)
        s2_ref[rows, :] = jnp.dot(
            h.astype(bf), w2_ref[...].astype(bf),
            preferred_element_type=jnp.float32).astype(bf)

    @pl.when(l == 1)
    def _phase1():
        agg = jnp.dot(adjbf_ref[rows, :], s2_ref[...],
                      preferred_element_type=jnp.float32)
        h = jnp.maximum(agg + b2_ref[...], 0.0)
        s3_ref[rows, :] = jnp.dot(
            h.astype(bf), w3_ref[...].astype(bf),
            preferred_element_type=jnp.float32).astype(bf)

    @pl.when(l == 2)
    def _phase2():
        agg = jnp.dot(adjbf_ref[rows, :], s3_ref[...],
                      preferred_element_type=jnp.float32)
        o_ref[...] = jnp.maximum(agg + b3_ref[...], 0.0)


def kernel(x, adj, W1, b1, W2, b2, W3, b3):
    n, d_in = x.shape
    c1 = W1.shape[1]
    c2 = W2.shape[1]
    c3 = W3.shape[1]
    nb = n // _BM

    return pl.pallas_call(
        _gcn_body,
        grid=(3, nb),
        in_specs=[
            pl.BlockSpec((n, d_in), lambda l, i: (0, 0)),
            pl.BlockSpec((_BM, n), lambda l, i: (jnp.where(l == 0, i, 0), 0)),
            pl.BlockSpec((d_in, c1), lambda l, i: (0, 0)),
            pl.BlockSpec((c1, c2), lambda l, i: (0, 0)),
            pl.BlockSpec((c2, c3), lambda l, i: (0, 0)),
            pl.BlockSpec((1, c1), lambda l, i: (0, 0)),
            pl.BlockSpec((1, c2), lambda l, i: (0, 0)),
            pl.BlockSpec((1, c3), lambda l, i: (0, 0)),
        ],
        out_specs=pl.BlockSpec((_BM, c3), lambda l, i: (jnp.where(l == 2, i, 0), 0)),
        out_shape=jax.ShapeDtypeStruct((n, c3), jnp.float32),
        scratch_shapes=[
            pltpu.VMEM((n, n), jnp.bfloat16),
            pltpu.VMEM((n, c2), jnp.bfloat16),
            pltpu.VMEM((n, c3), jnp.bfloat16),
        ],
    )(x.astype(jnp.bfloat16), adj, W1, W2, W3,
      b1.reshape(1, -1), b2.reshape(1, -1), b3.reshape(1, -1))


# speedup vs baseline: 1.4455x; 1.3130x over previous
import jax
import jax.numpy as jnp
from jax.experimental import pallas as pl
from jax.experimental.pallas import tpu as pltpu

_BM = 256


def _gcn_body(x_ref, adj_ref, w1_ref, w2_ref, w3_ref, b1_ref, b2_ref, b3_ref,
              o_ref, adjbf_ref, s2_ref, s3_ref):
    l = pl.program_id(0)
    i = pl.program_id(1)
    bf = jnp.bfloat16
    rows = pl.ds(i * _BM, _BM)

    @pl.when(l == 0)
    def _phase0():
        ab = adj_ref[...].astype(bf)
        adjbf_ref[rows, :] = ab
        aggx = jnp.dot(ab, x_ref[...], preferred_element_type=jnp.float32)
        t = jnp.dot(aggx.astype(bf), w1_ref[...].astype(bf),
                    preferred_element_type=jnp.float32)
        h = jnp.maximum(t + b1_ref[...], 0.0)
        s2_ref[rows, :] = jnp.dot(
            h.astype(bf), w2_ref[...].astype(bf),
            preferred_element_type=jnp.float32).astype(bf)

    @pl.when(l == 1)
    def _phase1():
        agg = jnp.dot(adjbf_ref[rows, :], s2_ref[...],
                      preferred_element_type=jnp.float32)
        h = jnp.maximum(agg + b2_ref[...], 0.0)
        s3_ref[rows, :] = jnp.dot(
            h.astype(bf), w3_ref[...].astype(bf),
            preferred_element_type=jnp.float32).astype(bf)

    @pl.when(l == 2)
    def _phase2():
        agg = jnp.dot(adjbf_ref[rows, :], s3_ref[...],
                      preferred_element_type=jnp.float32)
        o_ref[...] = jnp.maximum(agg + b3_ref[...], 0.0)


def kernel(x, adj, W1, b1, W2, b2, W3, b3):
    n, d_in = x.shape
    c1 = W1.shape[1]
    c2 = W2.shape[1]
    c3 = W3.shape[1]
    nb = n // _BM

    return pl.pallas_call(
        _gcn_body,
        grid=(3, nb),
        in_specs=[
            pl.BlockSpec((n, d_in), lambda l, i: (0, 0)),
            pl.BlockSpec((_BM, n), lambda l, i: (jnp.where(l == 0, i, 0), 0)),
            pl.BlockSpec((d_in, c1), lambda l, i: (0, 0)),
            pl.BlockSpec((c1, c2), lambda l, i: (0, 0)),
            pl.BlockSpec((c2, c3), lambda l, i: (0, 0)),
            pl.BlockSpec((1, c1), lambda l, i: (0, 0)),
            pl.BlockSpec((1, c2), lambda l, i: (0, 0)),
            pl.BlockSpec((1, c3), lambda l, i: (0, 0)),
        ],
        out_specs=pl.BlockSpec((_BM, c3), lambda l, i: (jnp.where(l == 2, i, 0), 0)),
        out_shape=jax.ShapeDtypeStruct((n, c3), jnp.float32),
        scratch_shapes=[
            pltpu.VMEM((n, n), jnp.bfloat16),
            pltpu.VMEM((n, c2), jnp.bfloat16),
            pltpu.VMEM((n, c3), jnp.bfloat16),
        ],
    )(x.astype(jnp.bfloat16), adj, W1, W2, W3,
      b1.reshape(1, -1), b2.reshape(1, -1), b3.reshape(1, -1))


# speedup vs baseline: 1.5703x; 1.0864x over previous
import jax
import jax.numpy as jnp
from jax.experimental import pallas as pl
from jax.experimental.pallas import tpu as pltpu

_B0 = 256
_B12 = 512
_N0 = 16
_N12 = 8


def _gcn_body(x_ref, adj_ref, w1_ref, w2_ref, w3_ref, b1_ref, b2_ref, b3_ref,
              o_ref, adjbf_ref, s2_ref, s3_ref):
    t = pl.program_id(0)
    bf = jnp.bfloat16

    @pl.when(t < _N0)
    def _phase0():
        rows = pl.ds(t * _B0, _B0)
        ab = adj_ref[...].astype(bf)
        adjbf_ref[rows, :] = ab
        aggx = jnp.dot(ab, x_ref[...], preferred_element_type=jnp.float32)
        z = jnp.dot(aggx.astype(bf), w1_ref[...].astype(bf),
                    preferred_element_type=jnp.float32)
        h = jnp.maximum(z + b1_ref[...], 0.0)
        s2_ref[rows, :] = jnp.dot(
            h.astype(bf), w2_ref[...].astype(bf),
            preferred_element_type=jnp.float32).astype(bf)

    @pl.when((t >= _N0) & (t < _N0 + _N12))
    def _phase1():
        rows = pl.ds((t - _N0) * _B12, _B12)
        agg = jnp.dot(adjbf_ref[rows, :], s2_ref[...],
                      preferred_element_type=jnp.float32)
        h = jnp.maximum(agg + b2_ref[...], 0.0)
        s3_ref[rows, :] = jnp.dot(
            h.astype(bf), w3_ref[...].astype(bf),
            preferred_element_type=jnp.float32).astype(bf)

    @pl.when(t >= _N0 + _N12)
    def _phase2():
        rows = pl.ds((t - _N0 - _N12) * _B12, _B12)
        agg = jnp.dot(adjbf_ref[rows, :], s3_ref[...],
                      preferred_element_type=jnp.float32)
        o_ref[...] = jnp.maximum(agg + b3_ref[...], 0.0)


def kernel(x, adj, W1, b1, W2, b2, W3, b3):
    n, d_in = x.shape
    c1 = W1.shape[1]
    c2 = W2.shape[1]
    c3 = W3.shape[1]

    return pl.pallas_call(
        _gcn_body,
        grid=(_N0 + 2 * _N12,),
        in_specs=[
            pl.BlockSpec((n, d_in), lambda t: (0, 0)),
            pl.BlockSpec((_B0, n), lambda t: (jnp.where(t < _N0, t, 0), 0)),
            pl.BlockSpec((d_in, c1), lambda t: (0, 0)),
            pl.BlockSpec((c1, c2), lambda t: (0, 0)),
            pl.BlockSpec((c2, c3), lambda t: (0, 0)),
            pl.BlockSpec((1, c1), lambda t: (0, 0)),
            pl.BlockSpec((1, c2), lambda t: (0, 0)),
            pl.BlockSpec((1, c3), lambda t: (0, 0)),
        ],
        out_specs=pl.BlockSpec(
            (_B12, c3),
            lambda t: (jnp.where(t >= _N0 + _N12, t - _N0 - _N12, 0), 0)),
        out_shape=jax.ShapeDtypeStruct((n, c3), jnp.float32),
        scratch_shapes=[
            pltpu.VMEM((n, n), jnp.bfloat16),
            pltpu.VMEM((n, c2), jnp.bfloat16),
            pltpu.VMEM((n, c3), jnp.bfloat16),
        ],
    )(x.astype(jnp.bfloat16), adj, W1, W2, W3,
      b1.reshape(1, -1), b2.reshape(1, -1), b3.reshape(1, -1))
